# initial kernel scaffold (unmeasured)
import jax
import jax.numpy as jnp
from jax import lax
from jax.experimental import pallas as pl
from jax.experimental.pallas import tpu as pltpu

NBITS = 13


def kernel(x, dest):
    T, D = x.shape
    my_y_out = lax.axis_index("y")

    send_mask = (dest != my_y_out).astype(jnp.int32)
    perm = jnp.argsort(send_mask, stable=True)
    x_sorted = jnp.take(x, perm, axis=0)
    n_keep_arr = jnp.sum(1 - send_mask).astype(jnp.int32).reshape(1)

    def body(n_keep_ref, x_ref, out_ref, send_sems, recv_sems, copy_sems):
        my_x = lax.axis_index("x")
        my_y = lax.axis_index("y")
        my_z = lax.axis_index("z")
        peer = (my_x, 1 - my_y, my_z)

        n_keep = n_keep_ref[0]
        n_move = T - n_keep

        keep_base = jnp.where(my_y == 0, 0, n_move)
        dst_base = jnp.where(my_y == 0, 0, n_keep)
        recv_base = jnp.where(my_y == 0, n_keep, 0)

        barrier_sem = pltpu.get_barrier_semaphore()
        pl.semaphore_signal(
            barrier_sem, inc=1, device_id=peer,
            device_id_type=pl.DeviceIdType.MESH,
        )
        pl.semaphore_wait(barrier_sem, 1)

        def off_of(count, b):
            return (count >> (b + 1)) << (b + 1)

        for b in reversed(range(NBITS)):
            size = 1 << b

            @pl.when((n_keep >> b) & 1 == 1)
            def _(b=b, size=size):
                off = off_of(n_keep, b)
                pltpu.make_async_copy(
                    x_ref.at[pl.ds(off, size)],
                    out_ref.at[pl.ds(keep_base + off, size)],
                    copy_sems.at[b],
                ).start()

            @pl.when((n_move >> b) & 1 == 1)
            def _(b=b, size=size):
                off = off_of(n_move, b)
                pltpu.make_async_remote_copy(
                    src_ref=x_ref.at[pl.ds(n_keep + off, size)],
                    dst_ref=out_ref.at[pl.ds(dst_base + off, size)],
                    send_sem=send_sems.at[b],
                    recv_sem=recv_sems.at[b],
                    device_id=peer,
                    device_id_type=pl.DeviceIdType.MESH,
                ).start()

        for b in range(NBITS):
            size = 1 << b

            @pl.when((n_keep >> b) & 1 == 1)
            def _(b=b, size=size):
                off = off_of(n_keep, b)
                pltpu.make_async_copy(
                    x_ref.at[pl.ds(off, size)],
                    out_ref.at[pl.ds(keep_base + off, size)],
                    copy_sems.at[b],
                ).wait()

            @pl.when((n_move >> b) & 1 == 1)
            def _(b=b, size=size):
                off = off_of(n_move, b)
                desc = pltpu.make_async_remote_copy(
                    src_ref=x_ref.at[pl.ds(n_keep + off, size)],
                    dst_ref=out_ref.at[pl.ds(recv_base + off, size)],
                    send_sem=send_sems.at[b],
                    recv_sem=recv_sems.at[b],
                    device_id=peer,
                    device_id_type=pl.DeviceIdType.MESH,
                )
                desc.wait_send()
                desc.wait_recv()

    return pl.pallas_call(
        body,
        out_shape=jax.ShapeDtypeStruct((T, D), jnp.float32),
        in_specs=[
            pl.BlockSpec(memory_space=pltpu.SMEM),
            pl.BlockSpec(memory_space=pltpu.ANY),
        ],
        out_specs=pl.BlockSpec(memory_space=pltpu.ANY),
        scratch_shapes=[
            pltpu.SemaphoreType.DMA((NBITS,)),
            pltpu.SemaphoreType.DMA((NBITS,)),
            pltpu.SemaphoreType.DMA((NBITS,)),
        ],
        compiler_params=pltpu.CompilerParams(collective_id=0),
    )(n_keep_arr, x_sorted)


# baseline (device time: 277791 ns/iter reference)
import jax
import jax.numpy as jnp
from jax import lax
from jax.experimental import pallas as pl
from jax.experimental.pallas import tpu as pltpu

NBITS = 13


def kernel(x, dest):
    T, D = x.shape
    my_y_out = lax.axis_index("y")

    send_mask = (dest != my_y_out).astype(jnp.int32)
    perm = jnp.argsort(send_mask, stable=True)
    x_sorted = jnp.take(x, perm, axis=0)
    n_keep_arr = jnp.sum(1 - send_mask).astype(jnp.int32).reshape(1)

    R = D // 128
    assert R % 8 == 0
    x_sorted = x_sorted.reshape(T * R, 128)

    def body(n_keep_ref, x_ref, out_ref, send_sems, recv_sems, copy_sems):
        my_x = lax.axis_index("x")
        my_y = lax.axis_index("y")
        my_z = lax.axis_index("z")
        peer = (my_x, 1 - my_y, my_z)

        n_keep = n_keep_ref[0]
        n_move = T - n_keep

        keep_base = jnp.where(my_y == 0, 0, n_move)
        dst_base = jnp.where(my_y == 0, 0, n_keep)
        recv_base = jnp.where(my_y == 0, n_keep, 0)

        barrier_sem = pltpu.get_barrier_semaphore()
        pl.semaphore_signal(
            barrier_sem, inc=1, device_id=peer,
            device_id_type=pl.DeviceIdType.MESH,
        )
        pl.semaphore_wait(barrier_sem, 1)

        def off_of(count, b):
            return (count >> (b + 1)) << (b + 1)

        for b in reversed(range(NBITS)):
            size = 1 << b

            @pl.when((n_keep >> b) & 1 == 1)
            def _(b=b, size=size):
                off = off_of(n_keep, b)
                pltpu.make_async_copy(
                    x_ref.at[pl.ds(off * R, size * R)],
                    out_ref.at[pl.ds((keep_base + off) * R, size * R)],
                    copy_sems.at[b],
                ).start()

            @pl.when((n_move >> b) & 1 == 1)
            def _(b=b, size=size):
                off = off_of(n_move, b)
                pltpu.make_async_remote_copy(
                    src_ref=x_ref.at[pl.ds((n_keep + off) * R, size * R)],
                    dst_ref=out_ref.at[pl.ds((dst_base + off) * R, size * R)],
                    send_sem=send_sems.at[b],
                    recv_sem=recv_sems.at[b],
                    device_id=peer,
                    device_id_type=pl.DeviceIdType.MESH,
                ).start()

        for b in range(NBITS):
            size = 1 << b

            @pl.when((n_keep >> b) & 1 == 1)
            def _(b=b, size=size):
                off = off_of(n_keep, b)
                pltpu.make_async_copy(
                    x_ref.at[pl.ds(off * R, size * R)],
                    out_ref.at[pl.ds((keep_base + off) * R, size * R)],
                    copy_sems.at[b],
                ).wait()

            @pl.when((n_move >> b) & 1 == 1)
            def _(b=b, size=size):
                off = off_of(n_move, b)
                desc = pltpu.make_async_remote_copy(
                    src_ref=x_ref.at[pl.ds((n_keep + off) * R, size * R)],
                    dst_ref=out_ref.at[pl.ds((recv_base + off) * R, size * R)],
                    send_sem=send_sems.at[b],
                    recv_sem=recv_sems.at[b],
                    device_id=peer,
                    device_id_type=pl.DeviceIdType.MESH,
                )
                desc.wait_send()
                desc.wait_recv()

    out = pl.pallas_call(
        body,
        out_shape=jax.ShapeDtypeStruct((T * R, 128), jnp.float32),
        in_specs=[
            pl.BlockSpec(memory_space=pltpu.SMEM),
            pl.BlockSpec(memory_space=pl.ANY),
        ],
        out_specs=pl.BlockSpec(memory_space=pl.ANY),
        scratch_shapes=[
            pltpu.SemaphoreType.DMA((NBITS,)),
            pltpu.SemaphoreType.DMA((NBITS,)),
            pltpu.SemaphoreType.DMA((NBITS,)),
        ],
        compiler_params=pltpu.CompilerParams(collective_id=0),
    )(n_keep_arr, x_sorted)
    return out.reshape(T, D)


# device time: 143374 ns/iter; 1.9375x vs baseline; 1.9375x over previous
import jax
import jax.numpy as jnp
from jax import lax
from jax.experimental import pallas as pl
from jax.experimental.pallas import tpu as pltpu

NBITS = 13


def kernel(x, dest):
    T, D = x.shape
    R = D // 128
    my_y_out = lax.axis_index("y")

    send_mask = (dest != my_y_out).astype(jnp.int32)
    perm = jnp.argsort(send_mask, stable=True).astype(jnp.int32)
    n_keep_arr = jnp.sum(1 - send_mask).astype(jnp.int32).reshape(1)

    x3 = x.reshape(T, R, 128)

    def body(n_keep_ref, perm_ref, x_ref, out_ref, send_buf, send_sems, recv_sems):
        my_x = lax.axis_index("x")
        my_y = lax.axis_index("y")
        my_z = lax.axis_index("z")
        peer = (my_x, 1 - my_y, my_z)

        n_keep = n_keep_ref[0]
        n_move = T - n_keep

        keep_base = jnp.where(my_y == 0, 0, n_move)
        dst_base = jnp.where(my_y == 0, 0, n_keep)
        recv_base = jnp.where(my_y == 0, n_keep, 0)

        barrier_sem = pltpu.get_barrier_semaphore()
        pl.semaphore_signal(
            barrier_sem, inc=1, device_id=peer,
            device_id_type=pl.DeviceIdType.MESH,
        )
        pl.semaphore_wait(barrier_sem, 1)

        def gather_send(i, _):
            send_buf[pl.ds(i, 1)] = x_ref[pl.ds(perm_ref[n_keep + i], 1)]
            return 0

        lax.fori_loop(0, n_move, gather_send, 0)

        def off_of(count, b):
            return (count >> (b + 1)) << (b + 1)

        for b in reversed(range(NBITS)):
            size = 1 << b

            @pl.when((n_move >> b) & 1 == 1)
            def _(b=b, size=size):
                off = off_of(n_move, b)
                pltpu.make_async_remote_copy(
                    src_ref=send_buf.at[pl.ds(off, size)],
                    dst_ref=out_ref.at[pl.ds(dst_base + off, size)],
                    send_sem=send_sems.at[b],
                    recv_sem=recv_sems.at[b],
                    device_id=peer,
                    device_id_type=pl.DeviceIdType.MESH,
                ).start()

        def gather_keep(i, _):
            out_ref[pl.ds(keep_base + i, 1)] = x_ref[pl.ds(perm_ref[i], 1)]
            return 0

        lax.fori_loop(0, n_keep, gather_keep, 0)

        for b in range(NBITS):
            size = 1 << b

            @pl.when((n_move >> b) & 1 == 1)
            def _(b=b, size=size):
                off = off_of(n_move, b)
                desc = pltpu.make_async_remote_copy(
                    src_ref=send_buf.at[pl.ds(off, size)],
                    dst_ref=out_ref.at[pl.ds(recv_base + off, size)],
                    send_sem=send_sems.at[b],
                    recv_sem=recv_sems.at[b],
                    device_id=peer,
                    device_id_type=pl.DeviceIdType.MESH,
                )
                desc.wait_send()
                desc.wait_recv()

    out = pl.pallas_call(
        body,
        out_shape=jax.ShapeDtypeStruct((T, R, 128), jnp.float32),
        in_specs=[
            pl.BlockSpec(memory_space=pltpu.SMEM),
            pl.BlockSpec(memory_space=pltpu.SMEM),
            pl.BlockSpec(memory_space=pltpu.VMEM),
        ],
        out_specs=pl.BlockSpec(memory_space=pltpu.VMEM),
        scratch_shapes=[
            pltpu.VMEM((T, R, 128), jnp.float32),
            pltpu.SemaphoreType.DMA((NBITS,)),
            pltpu.SemaphoreType.DMA((NBITS,)),
        ],
        compiler_params=pltpu.CompilerParams(collective_id=0),
    )(n_keep_arr, perm, x3)
    return out.reshape(T, D)


# device time: 130882 ns/iter; 2.1225x vs baseline; 1.0954x over previous
import jax
import jax.numpy as jnp
from jax import lax
from jax.experimental import pallas as pl
from jax.experimental.pallas import tpu as pltpu

NBITS = 13


def kernel(x, dest):
    T, D = x.shape
    R = D // 128
    my_y_out = lax.axis_index("y")

    send_mask = (dest != my_y_out).astype(jnp.int32)
    perm = jnp.argsort(send_mask, stable=True).astype(jnp.int32)
    n_keep_arr = jnp.sum(1 - send_mask).astype(jnp.int32).reshape(1)

    x3 = x.reshape(T, R, 128)

    def body(n_keep_ref, perm_ref, x_ref, out_ref, send_buf, send_sems, recv_sems):
        my_x = lax.axis_index("x")
        my_y = lax.axis_index("y")
        my_z = lax.axis_index("z")
        peer = (my_x, 1 - my_y, my_z)

        n_keep = n_keep_ref[0]
        n_move = T - n_keep

        keep_base = jnp.where(my_y == 0, 0, n_move)
        dst_base = jnp.where(my_y == 0, 0, n_keep)
        recv_base = jnp.where(my_y == 0, n_keep, 0)

        barrier_sem = pltpu.get_barrier_semaphore()
        pl.semaphore_signal(
            barrier_sem, inc=1, device_id=peer,
            device_id_type=pl.DeviceIdType.MESH,
        )
        pl.semaphore_wait(barrier_sem, 1)

        for b in range(NBITS):
            size = 1 << b

            @pl.when((n_move >> b) & 1 == 1)
            def _(b=b, size=size):
                off = n_move & (size - 1)

                def gather_send(i, _):
                    send_buf[pl.ds(off + i, 1)] = x_ref[
                        pl.ds(perm_ref[n_keep + off + i], 1)
                    ]
                    return 0

                lax.fori_loop(0, size, gather_send, 0)
                pltpu.make_async_remote_copy(
                    src_ref=send_buf.at[pl.ds(off, size)],
                    dst_ref=out_ref.at[pl.ds(dst_base + off, size)],
                    send_sem=send_sems.at[b],
                    recv_sem=recv_sems.at[b],
                    device_id=peer,
                    device_id_type=pl.DeviceIdType.MESH,
                ).start()

        def gather_keep(i, _):
            out_ref[pl.ds(keep_base + i, 1)] = x_ref[pl.ds(perm_ref[i], 1)]
            return 0

        lax.fori_loop(0, n_keep, gather_keep, 0)

        for b in range(NBITS):
            size = 1 << b

            @pl.when((n_move >> b) & 1 == 1)
            def _(b=b, size=size):
                off = n_move & (size - 1)
                desc = pltpu.make_async_remote_copy(
                    src_ref=send_buf.at[pl.ds(off, size)],
                    dst_ref=out_ref.at[pl.ds(recv_base + off, size)],
                    send_sem=send_sems.at[b],
                    recv_sem=recv_sems.at[b],
                    device_id=peer,
                    device_id_type=pl.DeviceIdType.MESH,
                )
                desc.wait_send()
                desc.wait_recv()

    out = pl.pallas_call(
        body,
        out_shape=jax.ShapeDtypeStruct((T, R, 128), jnp.float32),
        in_specs=[
            pl.BlockSpec(memory_space=pltpu.SMEM),
            pl.BlockSpec(memory_space=pltpu.SMEM),
            pl.BlockSpec(memory_space=pltpu.VMEM),
        ],
        out_specs=pl.BlockSpec(memory_space=pltpu.VMEM),
        scratch_shapes=[
            pltpu.VMEM((T, R, 128), jnp.float32),
            pltpu.SemaphoreType.DMA((NBITS,)),
            pltpu.SemaphoreType.DMA((NBITS,)),
        ],
        compiler_params=pltpu.CompilerParams(collective_id=0),
    )(n_keep_arr, perm, x3)
    return out.reshape(T, D)
